# baseline (speedup 1.0000x reference)
import functools

import jax
import jax.numpy as jnp
from jax import lax
from jax.experimental import pallas as pl
from jax.experimental.pallas import tpu as pltpu
from jax.experimental.pallas import tpu_sc as plsc

N = 10000
E = 320000
D = 128
U = 16

NC = 2
NS = 16
NW = NC * NS
CB = 128

NP_ = 10240
RPS = NP_ // NS
GB = 2 * CB
NCH = 2 * (-(-E // (NW * GB)))
NCG = NCH // 2
EP = NW * NCH * CB
PB = 128

_vmesh = plsc.VectorSubcoreMesh(core_axis_name="c", subcore_axis_name="s")
_sc_params = pltpu.CompilerParams(use_tc_tiling_on_sc=False)

_NPU = jax.ShapeDtypeStruct((NP_, U), jnp.float32)
_2NPU = jax.ShapeDtypeStruct((NC, NP_, U), jnp.float32)


def _edge_loop(table, gidx_v, sidx_v, w_s, acc, rows_v, sc_v,
               gs0, gs1, ss0, ss1, wid, w):
    rb = (rows_v.at[0], rows_v.at[1])
    sb = (sc_v.at[0], sc_v.at[1])
    gs = (gs0, gs1)
    ss = (ss0, ss1)

    def fire(p, b):
        if table is not None:
            pltpu.async_copy(table.at[gidx_v.at[p]], rb[b], gs[b])
        pltpu.async_copy(w.at[wid, p], w_s.at[b], gs[b])

    def scat_wait(p, b):
        h0 = sb[b].at[pl.ds(0, CB)]
        h1 = sb[b].at[pl.ds(CB, CB)]
        pltpu.make_async_copy(h0, acc.at[sidx_v.at[2 * p]], ss[b]).wait()
        pltpu.make_async_copy(h1, acc.at[sidx_v.at[2 * p + 1]], ss[b]).wait()

    def proc(p, b, first):
        if table is not None:
            pltpu.make_async_copy(table.at[gidx_v.at[p]], rb[b], gs[b]).wait()
        pltpu.make_async_copy(w.at[wid, p], w_s.at[b], gs[b]).wait()
        if not first:
            scat_wait(p - 2, b)

        @pl.loop(0, GB // U)
        def _(j):
            wv = w_s[b, pl.ds(j * U, U)]
            for i in range(U):
                r = j * U + i
                if table is not None:
                    sb[b][r, :] = rb[b][r, :] * wv[i]
                else:
                    sb[b][r, :] = lax.broadcast(wv[i], (U,))

        pltpu.async_copy(sb[b].at[pl.ds(0, CB)],
                         acc.at[sidx_v.at[2 * p]], ss[b], add=True)
        pltpu.async_copy(sb[b].at[pl.ds(CB, CB)],
                         acc.at[sidx_v.at[2 * p + 1]], ss[b], add=True)

    fire(0, 0)
    fire(1, 1)
    proc(0, 0, True)
    fire(2, 0)
    proc(1, 1, True)
    fire(3, 1)

    @pl.loop(0, (NCG - 2) // 2 - 1)
    def _(q):
        p = 2 * q + 2
        proc(p, 0, False)
        fire(p + 2, 0)
        proc(p + 1, 1, False)
        fire(p + 3, 1)

    proc(NCG - 2, 0, False)
    proc(NCG - 1, 1, False)
    scat_wait(NCG - 2, 0)
    scat_wait(NCG - 1, 1)


def _zero_acc(acc, sc_v, sid):
    @pl.loop(0, PB)
    def _(i):
        sc_v[0, i, :] = jnp.zeros((U,), jnp.float32)

    @pl.loop(0, RPS // PB)
    def _(j):
        pltpu.sync_copy(sc_v.at[0, pl.ds(0, PB)],
                        acc.at[pl.ds(sid * RPS + j * PB, PB)])


def _load4(refs_tiles, cs):
    for src, dst in refs_tiles:
        pltpu.async_copy(src, dst, cs)
    for src, dst in refs_tiles:
        pltpu.make_async_copy(src, dst, cs).wait()


def _build(parts, prev, ck, tdst, qdst, dis_own, comb_v, cs, cid, sid):
    @pl.loop(0, RPS // PB)
    def _(j):
        base = sid * RPS + j * PB
        _load4([(parts.at[0, pl.ds(base, PB)], comb_v.at[0]),
                (parts.at[1, pl.ds(base, PB)], comb_v.at[1]),
                (prev.at[pl.ds(base, PB)], comb_v.at[2]),
                (ck.at[pl.ds(base, PB)], comb_v.at[3])], cs)

        @pl.loop(0, PB, unroll=4)
        def _(i):
            dv = dis_own[j * PB + i, :]
            t = (dv * (comb_v[0, i, :] + comb_v[1, i, :])
                 + comb_v[2, i, :] + comb_v[3, i, :])
            comb_v[0, i, :] = t
            comb_v[1, i, :] = dv * t

        pltpu.sync_copy(comb_v.at[1], qdst.at[pl.ds(base, PB)])

        @pl.when(cid == 0)
        def _():
            pltpu.sync_copy(comb_v.at[0], tdst.at[pl.ds(base, PB)])


_SC_SCRATCH = [
    pltpu.VMEM((NCG, GB), jnp.int32),
    pltpu.VMEM((NCH, CB), jnp.int32),
    pltpu.VMEM((2, GB, U), jnp.float32),
    pltpu.VMEM((2, GB, U), jnp.float32),
    pltpu.VMEM((2, GB), jnp.float32),
    pltpu.VMEM((4, PB, U), jnp.float32),
    pltpu.VMEM((RPS, U), jnp.float32),
    pltpu.VMEM_SHARED((NP_, U), jnp.float32),
    pltpu.SemaphoreType.DMA,
    pltpu.SemaphoreType.DMA,
    pltpu.SemaphoreType.DMA,
    pltpu.SemaphoreType.DMA,
    pltpu.SemaphoreType.DMA,
    pltpu.SemaphoreType.REGULAR,
]


def _rsqrt_newton(d):
    dm = jnp.maximum(d, 1e-12)
    bits = lax.bitcast_convert_type(dm, jnp.int32)
    y = lax.bitcast_convert_type(
        jnp.int32(0x5F3759DF) - lax.shift_right_logical(bits, 1),
        jnp.float32)
    hx = 0.5 * dm
    for _ in range(3):
        y = y * (1.5 - hx * y * y)
    return jnp.where(d > 0, y, 0.0)


@functools.partial(
    pl.kernel,
    out_type=(_2NPU, _NPU, _NPU, _2NPU, _2NPU, _NPU, _2NPU, _2NPU),
    mesh=_vmesh,
    compiler_params=_sc_params,
    scratch_types=_SC_SCRATCH,
)
def _layer1(C, gidx, sidx, ridxs, w,
            parts, t_fin, dis16, qscr, degscr, tA, pA, pB,
            gidx_v, sidx_v, rows_v, sc_v, w_s, comb_v, dis_own, acc,
            gs0, gs1, ss0, ss1, cs, bar):
    cid = lax.axis_index("c")
    sid = lax.axis_index("s")
    wid = cid * NS + sid
    stripe = pl.ds(sid * RPS, RPS)

    pltpu.sync_copy(gidx.at[wid], gidx_v)
    pltpu.sync_copy(ridxs.at[wid], sidx_v)
    _zero_acc(acc, sc_v, sid)
    plsc.subcore_barrier()

    _edge_loop(None, gidx_v, sidx_v, w_s, acc, rows_v, sc_v,
               gs0, gs1, ss0, ss1, wid, w)
    pltpu.sync_copy(sidx.at[wid], sidx_v)
    plsc.subcore_barrier()
    pltpu.sync_copy(acc.at[stripe], degscr.at[cid, stripe])
    _zero_acc(acc, sc_v, sid)
    pltpu.core_barrier(bar, core_axis_name="c")

    @pl.loop(0, RPS // PB)
    def _(j):
        base = sid * RPS + j * PB
        _load4([(degscr.at[0, pl.ds(base, PB)], comb_v.at[0]),
                (degscr.at[1, pl.ds(base, PB)], comb_v.at[1]),
                (C.at[3, pl.ds(base, PB)], comb_v.at[2])], cs)

        @pl.loop(0, PB, unroll=4)
        def _(i):
            y = _rsqrt_newton(comb_v[0, i, :] + comb_v[1, i, :])
            dis_own[j * PB + i, :] = y
            comb_v[3, i, :] = y * comb_v[2, i, :]

        pltpu.sync_copy(comb_v.at[3], qscr.at[cid, pl.ds(base, PB)])

        @pl.when(cid == 0)
        def _():
            pltpu.sync_copy(dis_own.at[pl.ds(j * PB, PB)],
                            dis16.at[pl.ds(base, PB)])

    plsc.subcore_barrier()

    _edge_loop(qscr.at[cid], gidx_v, sidx_v, w_s, acc, rows_v, sc_v,
               gs0, gs1, ss0, ss1, wid, w)
    plsc.subcore_barrier()
    pltpu.sync_copy(acc.at[stripe], pA.at[cid, stripe])
    _zero_acc(acc, sc_v, sid)
    pltpu.core_barrier(bar, core_axis_name="c")

    _build(pA, C.at[3], C.at[2], tA, qscr.at[cid], dis_own, comb_v, cs,
           cid, sid)
    plsc.subcore_barrier()
    _edge_loop(qscr.at[cid], gidx_v, sidx_v, w_s, acc, rows_v, sc_v,
               gs0, gs1, ss0, ss1, wid, w)
    plsc.subcore_barrier()
    pltpu.sync_copy(acc.at[stripe], pB.at[cid, stripe])
    _zero_acc(acc, sc_v, sid)
    pltpu.core_barrier(bar, core_axis_name="c")

    _build(pB, tA, C.at[1], t_fin, qscr.at[cid], dis_own, comb_v, cs,
           cid, sid)
    plsc.subcore_barrier()
    _edge_loop(qscr.at[cid], gidx_v, sidx_v, w_s, acc, rows_v, sc_v,
               gs0, gs1, ss0, ss1, wid, w)
    plsc.subcore_barrier()
    pltpu.sync_copy(acc.at[stripe], parts.at[cid, stripe])


@functools.partial(
    pl.kernel,
    out_type=(_NPU, _2NPU, _NPU, _NPU, _2NPU, _2NPU, _2NPU),
    mesh=_vmesh,
    compiler_params=_sc_params,
    scratch_types=_SC_SCRATCH + [pltpu.VMEM((1, U), jnp.float32)],
)
def _layer2(Dm, dis16, b1r, gidx, sidx, w,
            out, qscr, uA, uB, pA, pB, pC,
            gidx_v, sidx_v, rows_v, sc_v, w_s, comb_v, dis_own, acc,
            gs0, gs1, ss0, ss1, cs, bar, b1_v):
    cid = lax.axis_index("c")
    sid = lax.axis_index("s")
    wid = cid * NS + sid
    stripe = pl.ds(sid * RPS, RPS)

    pltpu.sync_copy(gidx.at[wid], gidx_v)
    pltpu.sync_copy(sidx.at[wid], sidx_v)
    pltpu.sync_copy(dis16.at[stripe], dis_own)
    pltpu.sync_copy(b1r, b1_v)
    _zero_acc(acc, sc_v, sid)

    @pl.loop(0, RPS // PB)
    def _(j):
        base = sid * RPS + j * PB
        _load4([(Dm.at[3, pl.ds(base, PB)], comb_v.at[0])], cs)

        @pl.loop(0, PB, unroll=4)
        def _(i):
            comb_v[1, i, :] = dis_own[j * PB + i, :] * comb_v[0, i, :]

        pltpu.sync_copy(comb_v.at[1], qscr.at[cid, pl.ds(base, PB)])

    plsc.subcore_barrier()

    _edge_loop(qscr.at[cid], gidx_v, sidx_v, w_s, acc, rows_v, sc_v,
               gs0, gs1, ss0, ss1, wid, w)
    plsc.subcore_barrier()
    pltpu.sync_copy(acc.at[stripe], pA.at[cid, stripe])
    _zero_acc(acc, sc_v, sid)
    pltpu.core_barrier(bar, core_axis_name="c")

    _build(pA, Dm.at[3], Dm.at[2], uA, qscr.at[cid], dis_own, comb_v, cs,
           cid, sid)
    plsc.subcore_barrier()
    _edge_loop(qscr.at[cid], gidx_v, sidx_v, w_s, acc, rows_v, sc_v,
               gs0, gs1, ss0, ss1, wid, w)
    plsc.subcore_barrier()
    pltpu.sync_copy(acc.at[stripe], pB.at[cid, stripe])
    _zero_acc(acc, sc_v, sid)
    pltpu.core_barrier(bar, core_axis_name="c")

    _build(pB, uA, Dm.at[1], uB, qscr.at[cid], dis_own, comb_v, cs,
           cid, sid)
    plsc.subcore_barrier()
    _edge_loop(qscr.at[cid], gidx_v, sidx_v, w_s, acc, rows_v, sc_v,
               gs0, gs1, ss0, ss1, wid, w)
    plsc.subcore_barrier()
    pltpu.sync_copy(acc.at[stripe], pC.at[cid, stripe])
    pltpu.core_barrier(bar, core_axis_name="c")

    @pl.when(cid == 0)
    def _():
        @pl.loop(0, RPS // PB)
        def _(j):
            base = sid * RPS + j * PB
            _load4([(pC.at[0, pl.ds(base, PB)], comb_v.at[0]),
                    (pC.at[1, pl.ds(base, PB)], comb_v.at[1]),
                    (uB.at[pl.ds(base, PB)], comb_v.at[2]),
                    (Dm.at[0, pl.ds(base, PB)], comb_v.at[3])], cs)
            bv = b1_v[0, :]

            @pl.loop(0, PB, unroll=4)
            def _(i):
                comb_v[0, i, :] = (
                    dis_own[j * PB + i, :]
                    * (comb_v[0, i, :] + comb_v[1, i, :])
                    + comb_v[2, i, :] + comb_v[3, i, :] + bv)

            pltpu.sync_copy(comb_v.at[0], out.at[pl.ds(base, PB)])


def _mm1_body(x_ref, w_ref, o_ref):
    o_ref[...] = jnp.dot(x_ref[...], w_ref[...],
                         preferred_element_type=jnp.float32)


def _mm1(xp, w):
    return pl.pallas_call(
        _mm1_body,
        grid=(NP_ // 1024,),
        in_specs=[pl.BlockSpec((1024, D), lambda i: (i, 0)),
                  pl.BlockSpec((D, 4 * U), lambda i: (0, 0))],
        out_specs=pl.BlockSpec((1024, 4 * U), lambda i: (i, 0)),
        out_shape=jax.ShapeDtypeStruct((NP_, 4 * U), jnp.float32),
    )(xp, w)


def _mm2_body(p0_ref, p1_ref, prev_ref, ck_ref, dis_ref, b_ref, w_ref, o_ref):
    z = (dis_ref[...] * (p0_ref[...] + p1_ref[...])
         + prev_ref[...] + ck_ref[...])
    h = jnp.maximum(z + b_ref[...], 0.0)
    o_ref[...] = jnp.dot(h, w_ref[...], preferred_element_type=jnp.float32)


def _mm2(p0, p1, prev, ck, dis16, b0, w):
    nspec = pl.BlockSpec((1024, U), lambda i: (i, 0))
    return pl.pallas_call(
        _mm2_body,
        grid=(NP_ // 1024,),
        in_specs=[nspec, nspec, nspec, nspec, nspec,
                  pl.BlockSpec((1, U), lambda i: (0, 0)),
                  pl.BlockSpec((U, 4 * U), lambda i: (0, 0))],
        out_specs=pl.BlockSpec((1024, 4 * U), lambda i: (i, 0)),
        out_shape=jax.ShapeDtypeStruct((NP_, 4 * U), jnp.float32),
    )(p0, p1, prev, ck, dis16, b0, w)


def kernel(x, edge_index, edge_weight, W0, b0, W1, b1):
    row = edge_index[0]
    col = edge_index[1]
    pad_e = EP - E
    rowg = jnp.concatenate(
        [row, jnp.zeros((pad_e,), jnp.int32)]).reshape(NW, NCG, GB)
    rows_ = jnp.concatenate(
        [row, jnp.ones((pad_e,), jnp.int32)]).reshape(NW, NCH, CB)
    colp = jnp.concatenate(
        [col, jnp.zeros((pad_e,), jnp.int32)]).reshape(NW, NCH, CB)
    wp = jnp.concatenate(
        [edge_weight, jnp.zeros((pad_e,), jnp.float32)]).reshape(NW, NCG, GB)

    xp = jnp.pad(x, ((0, NP_ - N), (0, 0)))
    W0c = jnp.concatenate([W0[k * D:(k + 1) * D] for k in range(4)], axis=1)
    C = _mm1(xp, W0c)
    Csp = jnp.transpose(C.reshape(NP_, 4, U), (1, 0, 2))
    parts, t1v, dis16 = _layer1(Csp, rowg, colp, rows_, wp)[:3]

    W1c = jnp.concatenate([W1[k * U:(k + 1) * U] for k in range(4)], axis=1)
    Dm = _mm2(parts[0], parts[1], t1v, Csp[0], dis16,
              b0.reshape(1, U), W1c)
    Dsp = jnp.transpose(Dm.reshape(NP_, 4, U), (1, 0, 2))
    out = _layer2(Dsp, dis16, b1.reshape(1, U), rowg, colp, wp)[0]
    return out[:N]


# speedup vs baseline: 1.0035x; 1.0035x over previous
import functools

import jax
import jax.numpy as jnp
from jax import lax
from jax.experimental import pallas as pl
from jax.experimental.pallas import tpu as pltpu
from jax.experimental.pallas import tpu_sc as plsc

N = 10000
E = 320000
D = 128
U = 16

NC = 2
NS = 16
NW = NC * NS
CB = 128

NP_ = 10240
RPS = NP_ // NS
NCH = -(-E // (NW * CB))
EP = NW * NCH * CB
PB = 128

_vmesh = plsc.VectorSubcoreMesh(core_axis_name="c", subcore_axis_name="s")
_sc_params = pltpu.CompilerParams(use_tc_tiling_on_sc=False)

_NPU = jax.ShapeDtypeStruct((NP_, U), jnp.float32)
_2NPU = jax.ShapeDtypeStruct((NC, NP_, U), jnp.float32)


def _edge_loop(table, gidx_v, sidx_v, w_s, acc, rows_v, sc_v,
               gs0, gs1, ss0, ss1, wid, w):
    rb = (rows_v.at[0], rows_v.at[1])
    sb = (sc_v.at[0], sc_v.at[1])
    gs = (gs0, gs1)
    ss = (ss0, ss1)

    def fire(g, b):
        if table is not None:
            pltpu.async_copy(table.at[gidx_v.at[g]], rb[b], gs[b])
        pltpu.async_copy(w.at[wid, g], w_s.at[b], gs[b])

    def scat_wait(g, b):
        pltpu.make_async_copy(sb[b], acc.at[sidx_v.at[g]], ss[b]).wait()

    def proc(g, b, first):
        if table is not None:
            pltpu.make_async_copy(table.at[gidx_v.at[g]], rb[b], gs[b]).wait()
        pltpu.make_async_copy(w.at[wid, g], w_s.at[b], gs[b]).wait()
        if not first:
            scat_wait(g - 2, b)

        @pl.loop(0, CB // U, unroll=2)
        def _(j):
            wv = w_s[b, pl.ds(j * U, U)]
            for i in range(U):
                r = j * U + i
                if table is not None:
                    sb[b][r, :] = rb[b][r, :] * wv[i]
                else:
                    sb[b][r, :] = lax.broadcast(wv[i], (U,))

        pltpu.async_copy(sb[b], acc.at[sidx_v.at[g]], ss[b], add=True)

    fire(0, 0)
    fire(1, 1)
    proc(0, 0, True)
    fire(2, 0)
    proc(1, 1, True)
    fire(3, 1)

    @pl.loop(0, (NCH - 3) // 2)
    def _(p):
        g = 2 * p + 2
        proc(g, 0, False)
        fire(g + 2, 0)
        proc(g + 1, 1, False)

        @pl.when(g + 3 < NCH)
        def _():
            fire(g + 3, 1)

    proc(NCH - 1, 0, False)
    scat_wait(NCH - 2, 1)
    scat_wait(NCH - 1, 0)


def _zero_acc(acc, sc_v, sid):
    @pl.loop(0, PB)
    def _(i):
        sc_v[0, i, :] = jnp.zeros((U,), jnp.float32)

    @pl.loop(0, RPS // PB)
    def _(j):
        pltpu.sync_copy(sc_v.at[0, pl.ds(0, PB)],
                        acc.at[pl.ds(sid * RPS + j * PB, PB)])


def _load4(refs_tiles, cs):
    for src, dst in refs_tiles:
        pltpu.async_copy(src, dst, cs)
    for src, dst in refs_tiles:
        pltpu.make_async_copy(src, dst, cs).wait()


def _build(parts, prev, ck, tdst, qdst, dis_own, comb_v, cs, cid, sid):
    @pl.loop(0, RPS // PB)
    def _(j):
        base = sid * RPS + j * PB
        _load4([(parts.at[0, pl.ds(base, PB)], comb_v.at[0]),
                (parts.at[1, pl.ds(base, PB)], comb_v.at[1]),
                (prev.at[pl.ds(base, PB)], comb_v.at[2]),
                (ck.at[pl.ds(base, PB)], comb_v.at[3])], cs)

        @pl.loop(0, PB, unroll=4)
        def _(i):
            dv = dis_own[j * PB + i, :]
            t = (dv * (comb_v[0, i, :] + comb_v[1, i, :])
                 + comb_v[2, i, :] + comb_v[3, i, :])
            comb_v[0, i, :] = t
            comb_v[1, i, :] = dv * t

        pltpu.sync_copy(comb_v.at[1], qdst.at[pl.ds(base, PB)])

        @pl.when(cid == 0)
        def _():
            pltpu.sync_copy(comb_v.at[0], tdst.at[pl.ds(base, PB)])


_SC_SCRATCH = [
    pltpu.VMEM((NCH, CB), jnp.int32),
    pltpu.VMEM((NCH, CB), jnp.int32),
    pltpu.VMEM((2, CB, U), jnp.float32),
    pltpu.VMEM((2, CB, U), jnp.float32),
    pltpu.VMEM((2, CB), jnp.float32),
    pltpu.VMEM((4, PB, U), jnp.float32),
    pltpu.VMEM((RPS, U), jnp.float32),
    pltpu.VMEM_SHARED((NP_, U), jnp.float32),
    pltpu.SemaphoreType.DMA,
    pltpu.SemaphoreType.DMA,
    pltpu.SemaphoreType.DMA,
    pltpu.SemaphoreType.DMA,
    pltpu.SemaphoreType.DMA,
    pltpu.SemaphoreType.REGULAR,
]


def _rsqrt_newton(d):
    dm = jnp.maximum(d, 1e-12)
    bits = lax.bitcast_convert_type(dm, jnp.int32)
    y = lax.bitcast_convert_type(
        jnp.int32(0x5F3759DF) - lax.shift_right_logical(bits, 1),
        jnp.float32)
    hx = 0.5 * dm
    for _ in range(3):
        y = y * (1.5 - hx * y * y)
    return jnp.where(d > 0, y, 0.0)


@functools.partial(
    pl.kernel,
    out_type=(_2NPU, _NPU, _NPU, _2NPU, _2NPU, _NPU, _2NPU, _2NPU),
    mesh=_vmesh,
    compiler_params=_sc_params,
    scratch_types=_SC_SCRATCH,
)
def _layer1(C, gidx, sidx, w,
            parts, t_fin, dis16, qscr, degscr, tA, pA, pB,
            gidx_v, sidx_v, rows_v, sc_v, w_s, comb_v, dis_own, acc,
            gs0, gs1, ss0, ss1, cs, bar):
    cid = lax.axis_index("c")
    sid = lax.axis_index("s")
    wid = cid * NS + sid
    stripe = pl.ds(sid * RPS, RPS)

    pltpu.sync_copy(gidx.at[wid], gidx_v)
    pltpu.sync_copy(sidx.at[wid], sidx_v)
    _zero_acc(acc, sc_v, sid)
    plsc.subcore_barrier()

    _edge_loop(None, gidx_v, gidx_v, w_s, acc, rows_v, sc_v,
               gs0, gs1, ss0, ss1, wid, w)
    plsc.subcore_barrier()
    pltpu.sync_copy(acc.at[stripe], degscr.at[cid, stripe])
    _zero_acc(acc, sc_v, sid)
    pltpu.core_barrier(bar, core_axis_name="c")

    @pl.loop(0, RPS // PB)
    def _(j):
        base = sid * RPS + j * PB
        _load4([(degscr.at[0, pl.ds(base, PB)], comb_v.at[0]),
                (degscr.at[1, pl.ds(base, PB)], comb_v.at[1]),
                (C.at[3, pl.ds(base, PB)], comb_v.at[2])], cs)

        @pl.loop(0, PB, unroll=4)
        def _(i):
            y = _rsqrt_newton(comb_v[0, i, :] + comb_v[1, i, :])
            dis_own[j * PB + i, :] = y
            comb_v[3, i, :] = y * comb_v[2, i, :]

        pltpu.sync_copy(comb_v.at[3], qscr.at[cid, pl.ds(base, PB)])

        @pl.when(cid == 0)
        def _():
            pltpu.sync_copy(dis_own.at[pl.ds(j * PB, PB)],
                            dis16.at[pl.ds(base, PB)])

    plsc.subcore_barrier()

    _edge_loop(qscr.at[cid], gidx_v, sidx_v, w_s, acc, rows_v, sc_v,
               gs0, gs1, ss0, ss1, wid, w)
    plsc.subcore_barrier()
    pltpu.sync_copy(acc.at[stripe], pA.at[cid, stripe])
    _zero_acc(acc, sc_v, sid)
    pltpu.core_barrier(bar, core_axis_name="c")

    _build(pA, C.at[3], C.at[2], tA, qscr.at[cid], dis_own, comb_v, cs,
           cid, sid)
    plsc.subcore_barrier()
    _edge_loop(qscr.at[cid], gidx_v, sidx_v, w_s, acc, rows_v, sc_v,
               gs0, gs1, ss0, ss1, wid, w)
    plsc.subcore_barrier()
    pltpu.sync_copy(acc.at[stripe], pB.at[cid, stripe])
    _zero_acc(acc, sc_v, sid)
    pltpu.core_barrier(bar, core_axis_name="c")

    _build(pB, tA, C.at[1], t_fin, qscr.at[cid], dis_own, comb_v, cs,
           cid, sid)
    plsc.subcore_barrier()
    _edge_loop(qscr.at[cid], gidx_v, sidx_v, w_s, acc, rows_v, sc_v,
               gs0, gs1, ss0, ss1, wid, w)
    plsc.subcore_barrier()
    pltpu.sync_copy(acc.at[stripe], parts.at[cid, stripe])


@functools.partial(
    pl.kernel,
    out_type=(_NPU, _2NPU, _NPU, _NPU, _2NPU, _2NPU, _2NPU),
    mesh=_vmesh,
    compiler_params=_sc_params,
    scratch_types=_SC_SCRATCH + [pltpu.VMEM((1, U), jnp.float32)],
)
def _layer2(Dm, dis16, b1r, gidx, sidx, w,
            out, qscr, uA, uB, pA, pB, pC,
            gidx_v, sidx_v, rows_v, sc_v, w_s, comb_v, dis_own, acc,
            gs0, gs1, ss0, ss1, cs, bar, b1_v):
    cid = lax.axis_index("c")
    sid = lax.axis_index("s")
    wid = cid * NS + sid
    stripe = pl.ds(sid * RPS, RPS)

    pltpu.sync_copy(gidx.at[wid], gidx_v)
    pltpu.sync_copy(sidx.at[wid], sidx_v)
    pltpu.sync_copy(dis16.at[stripe], dis_own)
    pltpu.sync_copy(b1r, b1_v)
    _zero_acc(acc, sc_v, sid)

    @pl.loop(0, RPS // PB)
    def _(j):
        base = sid * RPS + j * PB
        _load4([(Dm.at[3, pl.ds(base, PB)], comb_v.at[0])], cs)

        @pl.loop(0, PB, unroll=4)
        def _(i):
            comb_v[1, i, :] = dis_own[j * PB + i, :] * comb_v[0, i, :]

        pltpu.sync_copy(comb_v.at[1], qscr.at[cid, pl.ds(base, PB)])

    plsc.subcore_barrier()

    _edge_loop(qscr.at[cid], gidx_v, sidx_v, w_s, acc, rows_v, sc_v,
               gs0, gs1, ss0, ss1, wid, w)
    plsc.subcore_barrier()
    pltpu.sync_copy(acc.at[stripe], pA.at[cid, stripe])
    _zero_acc(acc, sc_v, sid)
    pltpu.core_barrier(bar, core_axis_name="c")

    _build(pA, Dm.at[3], Dm.at[2], uA, qscr.at[cid], dis_own, comb_v, cs,
           cid, sid)
    plsc.subcore_barrier()
    _edge_loop(qscr.at[cid], gidx_v, sidx_v, w_s, acc, rows_v, sc_v,
               gs0, gs1, ss0, ss1, wid, w)
    plsc.subcore_barrier()
    pltpu.sync_copy(acc.at[stripe], pB.at[cid, stripe])
    _zero_acc(acc, sc_v, sid)
    pltpu.core_barrier(bar, core_axis_name="c")

    _build(pB, uA, Dm.at[1], uB, qscr.at[cid], dis_own, comb_v, cs,
           cid, sid)
    plsc.subcore_barrier()
    _edge_loop(qscr.at[cid], gidx_v, sidx_v, w_s, acc, rows_v, sc_v,
               gs0, gs1, ss0, ss1, wid, w)
    plsc.subcore_barrier()
    pltpu.sync_copy(acc.at[stripe], pC.at[cid, stripe])
    pltpu.core_barrier(bar, core_axis_name="c")

    @pl.when(cid == 0)
    def _():
        @pl.loop(0, RPS // PB)
        def _(j):
            base = sid * RPS + j * PB
            _load4([(pC.at[0, pl.ds(base, PB)], comb_v.at[0]),
                    (pC.at[1, pl.ds(base, PB)], comb_v.at[1]),
                    (uB.at[pl.ds(base, PB)], comb_v.at[2]),
                    (Dm.at[0, pl.ds(base, PB)], comb_v.at[3])], cs)
            bv = b1_v[0, :]

            @pl.loop(0, PB, unroll=4)
            def _(i):
                comb_v[0, i, :] = (
                    dis_own[j * PB + i, :]
                    * (comb_v[0, i, :] + comb_v[1, i, :])
                    + comb_v[2, i, :] + comb_v[3, i, :] + bv)

            pltpu.sync_copy(comb_v.at[0], out.at[pl.ds(base, PB)])


def _mm1_body(x_ref, w_ref, o_ref):
    o_ref[...] = jnp.dot(x_ref[...], w_ref[...],
                         preferred_element_type=jnp.float32)


def _mm1(xp, w):
    return pl.pallas_call(
        _mm1_body,
        grid=(NP_ // 1024,),
        in_specs=[pl.BlockSpec((1024, D), lambda i: (i, 0)),
                  pl.BlockSpec((D, 4 * U), lambda i: (0, 0))],
        out_specs=pl.BlockSpec((1024, 4 * U), lambda i: (i, 0)),
        out_shape=jax.ShapeDtypeStruct((NP_, 4 * U), jnp.float32),
    )(xp, w)


def _mm2_body(p0_ref, p1_ref, prev_ref, ck_ref, dis_ref, b_ref, w_ref, o_ref):
    z = (dis_ref[...] * (p0_ref[...] + p1_ref[...])
         + prev_ref[...] + ck_ref[...])
    h = jnp.maximum(z + b_ref[...], 0.0)
    o_ref[...] = jnp.dot(h, w_ref[...], preferred_element_type=jnp.float32)


def _mm2(p0, p1, prev, ck, dis16, b0, w):
    nspec = pl.BlockSpec((1024, U), lambda i: (i, 0))
    return pl.pallas_call(
        _mm2_body,
        grid=(NP_ // 1024,),
        in_specs=[nspec, nspec, nspec, nspec, nspec,
                  pl.BlockSpec((1, U), lambda i: (0, 0)),
                  pl.BlockSpec((U, 4 * U), lambda i: (0, 0))],
        out_specs=pl.BlockSpec((1024, 4 * U), lambda i: (i, 0)),
        out_shape=jax.ShapeDtypeStruct((NP_, 4 * U), jnp.float32),
    )(p0, p1, prev, ck, dis16, b0, w)


def kernel(x, edge_index, edge_weight, W0, b0, W1, b1):
    row = edge_index[0]
    col = edge_index[1]
    pad_e = EP - E
    rowp = jnp.concatenate(
        [row, jnp.zeros((pad_e,), jnp.int32)]).reshape(NW, NCH, CB)
    colp = jnp.concatenate(
        [col, jnp.zeros((pad_e,), jnp.int32)]).reshape(NW, NCH, CB)
    wp = jnp.concatenate(
        [edge_weight, jnp.zeros((pad_e,), jnp.float32)]).reshape(NW, NCH, CB)

    xp = jnp.pad(x, ((0, NP_ - N), (0, 0)))
    W0c = jnp.concatenate([W0[k * D:(k + 1) * D] for k in range(4)], axis=1)
    C = _mm1(xp, W0c)
    Csp = jnp.transpose(C.reshape(NP_, 4, U), (1, 0, 2))
    parts, t1v, dis16 = _layer1(Csp, rowp, colp, wp)[:3]

    W1c = jnp.concatenate([W1[k * U:(k + 1) * U] for k in range(4)], axis=1)
    Dm = _mm2(parts[0], parts[1], t1v, Csp[0], dis16,
              b0.reshape(1, U), W1c)
    Dsp = jnp.transpose(Dm.reshape(NP_, 4, U), (1, 0, 2))
    out = _layer2(Dsp, dis16, b1.reshape(1, U), rowp, colp, wp)[0]
    return out[:N]


# speedup vs baseline: 1.1546x; 1.1506x over previous
import functools

import jax
import jax.numpy as jnp
from jax import lax
from jax.experimental import pallas as pl
from jax.experimental.pallas import tpu as pltpu
from jax.experimental.pallas import tpu_sc as plsc

N = 10000
E = 320000
D = 128
U = 16

NC = 2
NS = 16
NW = NC * NS
CB = 128

NP_ = 10240
RPS = NP_ // NS
NCH = -(-E // (NW * CB))
EP = NW * NCH * CB
PB = 128

_vmesh = plsc.VectorSubcoreMesh(core_axis_name="c", subcore_axis_name="s")
_sc_params = pltpu.CompilerParams(use_tc_tiling_on_sc=False)

_NPU = jax.ShapeDtypeStruct((NP_, U), jnp.float32)
_2NPU = jax.ShapeDtypeStruct((NC, NP_, U), jnp.float32)


def _edge_loop(table, gidx_v, sidx_v, w_s, acc, rows_v, sc_v,
               gs0, gs1, ss0, ss1, wid, w):
    rb = (rows_v.at[0], rows_v.at[1])
    sb = (sc_v.at[0], sc_v.at[1])
    gs = (gs0, gs1)
    ss = (ss0, ss1)

    def fire(g, b):
        if table is not None:
            pltpu.async_copy(table.at[gidx_v.at[g]], rb[b], gs[b])
        pltpu.async_copy(w.at[wid, g], w_s.at[b], gs[b])

    def scat_wait(g, b):
        pltpu.make_async_copy(sb[b], acc.at[sidx_v.at[g]], ss[b]).wait()

    def proc(g, b, first):
        if table is not None:
            pltpu.make_async_copy(table.at[gidx_v.at[g]], rb[b], gs[b]).wait()
        pltpu.make_async_copy(w.at[wid, g], w_s.at[b], gs[b]).wait()
        if not first:
            scat_wait(g - 2, b)

        @pl.loop(0, CB // U)
        def _(j):
            wv = w_s[b, pl.ds(j * U, U)]
            for i in range(U):
                r = j * U + i
                if table is not None:
                    sb[b][r, :] = rb[b][r, :] * wv[i]
                else:
                    sb[b][r, :] = lax.broadcast(wv[i], (U,))

        pltpu.async_copy(sb[b], acc.at[sidx_v.at[g]], ss[b], add=True)

    fire(0, 0)
    fire(1, 1)
    proc(0, 0, True)
    fire(2, 0)
    proc(1, 1, True)
    fire(3, 1)

    @pl.loop(0, (NCH - 3) // 2)
    def _(p):
        g = 2 * p + 2
        proc(g, 0, False)
        fire(g + 2, 0)
        proc(g + 1, 1, False)

        @pl.when(g + 3 < NCH)
        def _():
            fire(g + 3, 1)

    proc(NCH - 1, 0, False)
    scat_wait(NCH - 2, 1)
    scat_wait(NCH - 1, 0)


def _zero_acc(acc, sc_v, sid):
    @pl.loop(0, PB)
    def _(i):
        sc_v[0, i, :] = jnp.zeros((U,), jnp.float32)

    @pl.loop(0, RPS // PB)
    def _(j):
        pltpu.sync_copy(sc_v.at[0, pl.ds(0, PB)],
                        acc.at[pl.ds(sid * RPS + j * PB, PB)])


def _load4(refs_tiles, cs):
    for src, dst in refs_tiles:
        pltpu.async_copy(src, dst, cs)
    for src, dst in refs_tiles:
        pltpu.make_async_copy(src, dst, cs).wait()


def _build(parts, prev, ck, tdst, qdst, dis_own, comb_v, cs, cid, sid):
    @pl.loop(0, RPS // PB)
    def _(j):
        base = sid * RPS + j * PB
        _load4([(parts.at[0, pl.ds(base, PB)], comb_v.at[0]),
                (parts.at[1, pl.ds(base, PB)], comb_v.at[1]),
                (prev.at[pl.ds(base, PB)], comb_v.at[2]),
                (ck.at[pl.ds(base, PB)], comb_v.at[3])], cs)

        @pl.loop(0, PB, unroll=4)
        def _(i):
            dv = dis_own[j * PB + i, :]
            t = (dv * (comb_v[0, i, :] + comb_v[1, i, :])
                 + comb_v[2, i, :] + comb_v[3, i, :])
            comb_v[0, i, :] = t
            comb_v[1, i, :] = dv * t

        pltpu.sync_copy(comb_v.at[1], qdst.at[pl.ds(base, PB)])

        @pl.when(cid == 0)
        def _():
            pltpu.sync_copy(comb_v.at[0], tdst.at[pl.ds(base, PB)])


_SC_SCRATCH = [
    pltpu.VMEM((NCH, CB), jnp.int32),
    pltpu.VMEM((NCH, CB), jnp.int32),
    pltpu.VMEM((2, CB, U), jnp.float32),
    pltpu.VMEM((2, CB, U), jnp.float32),
    pltpu.VMEM((2, CB), jnp.float32),
    pltpu.VMEM((4, PB, U), jnp.float32),
    pltpu.VMEM((RPS, U), jnp.float32),
    pltpu.VMEM_SHARED((NP_, U), jnp.float32),
    pltpu.SemaphoreType.DMA,
    pltpu.SemaphoreType.DMA,
    pltpu.SemaphoreType.DMA,
    pltpu.SemaphoreType.DMA,
    pltpu.SemaphoreType.DMA,
    pltpu.SemaphoreType.REGULAR,
]


def _rsqrt_newton(d):
    dm = jnp.maximum(d, 1e-12)
    bits = lax.bitcast_convert_type(dm, jnp.int32)
    y = lax.bitcast_convert_type(
        jnp.int32(0x5F3759DF) - lax.shift_right_logical(bits, 1),
        jnp.float32)
    hx = 0.5 * dm
    for _ in range(3):
        y = y * (1.5 - hx * y * y)
    return jnp.where(d > 0, y, 0.0)


@functools.partial(
    pl.kernel,
    out_type=(_2NPU, _NPU, _NPU, _2NPU, _2NPU, _NPU, _2NPU, _2NPU),
    mesh=_vmesh,
    compiler_params=_sc_params,
    scratch_types=_SC_SCRATCH,
)
def _layer1(C, gidx, sidx, w,
            parts, t_fin, dis16, qscr, degscr, tA, pA, pB,
            gidx_v, sidx_v, rows_v, sc_v, w_s, comb_v, dis_own, acc,
            gs0, gs1, ss0, ss1, cs, bar):
    cid = lax.axis_index("c")
    sid = lax.axis_index("s")
    wid = cid * NS + sid
    stripe = pl.ds(sid * RPS, RPS)

    pltpu.sync_copy(gidx.at[wid], gidx_v)
    pltpu.sync_copy(sidx.at[wid], sidx_v)
    _zero_acc(acc, sc_v, sid)
    plsc.subcore_barrier()

    _edge_loop(None, gidx_v, gidx_v, w_s, acc, rows_v, sc_v,
               gs0, gs1, ss0, ss1, wid, w)
    plsc.subcore_barrier()
    pltpu.sync_copy(acc.at[stripe], degscr.at[cid, stripe])
    _zero_acc(acc, sc_v, sid)
    pltpu.core_barrier(bar, core_axis_name="c")

    @pl.loop(0, RPS // PB)
    def _(j):
        base = sid * RPS + j * PB
        _load4([(degscr.at[0, pl.ds(base, PB)], comb_v.at[0]),
                (degscr.at[1, pl.ds(base, PB)], comb_v.at[1]),
                (C.at[3, pl.ds(base, PB)], comb_v.at[2])], cs)

        @pl.loop(0, PB, unroll=4)
        def _(i):
            y = _rsqrt_newton(comb_v[0, i, :] + comb_v[1, i, :])
            dis_own[j * PB + i, :] = y
            comb_v[3, i, :] = y * comb_v[2, i, :]

        pltpu.sync_copy(comb_v.at[3], qscr.at[cid, pl.ds(base, PB)])

        @pl.when(cid == 0)
        def _():
            pltpu.sync_copy(dis_own.at[pl.ds(j * PB, PB)],
                            dis16.at[pl.ds(base, PB)])

    plsc.subcore_barrier()

    _edge_loop(qscr.at[cid], gidx_v, sidx_v, w_s, acc, rows_v, sc_v,
               gs0, gs1, ss0, ss1, wid, w)
    plsc.subcore_barrier()
    pltpu.sync_copy(acc.at[stripe], pA.at[cid, stripe])
    _zero_acc(acc, sc_v, sid)
    pltpu.core_barrier(bar, core_axis_name="c")

    _build(pA, C.at[3], C.at[2], tA, qscr.at[cid], dis_own, comb_v, cs,
           cid, sid)
    plsc.subcore_barrier()
    _edge_loop(qscr.at[cid], gidx_v, sidx_v, w_s, acc, rows_v, sc_v,
               gs0, gs1, ss0, ss1, wid, w)
    plsc.subcore_barrier()
    pltpu.sync_copy(acc.at[stripe], pB.at[cid, stripe])
    _zero_acc(acc, sc_v, sid)
    pltpu.core_barrier(bar, core_axis_name="c")

    _build(pB, tA, C.at[1], t_fin, qscr.at[cid], dis_own, comb_v, cs,
           cid, sid)
    plsc.subcore_barrier()
    _edge_loop(qscr.at[cid], gidx_v, sidx_v, w_s, acc, rows_v, sc_v,
               gs0, gs1, ss0, ss1, wid, w)
    plsc.subcore_barrier()
    pltpu.sync_copy(acc.at[stripe], parts.at[cid, stripe])


@functools.partial(
    pl.kernel,
    out_type=(_NPU, _2NPU, _NPU, _NPU, _2NPU, _2NPU, _2NPU),
    mesh=_vmesh,
    compiler_params=_sc_params,
    scratch_types=_SC_SCRATCH + [pltpu.VMEM((1, U), jnp.float32)],
)
def _layer2(Dm, dis16, b1r, gidx, sidx, w,
            out, qscr, uA, uB, pA, pB, pC,
            gidx_v, sidx_v, rows_v, sc_v, w_s, comb_v, dis_own, acc,
            gs0, gs1, ss0, ss1, cs, bar, b1_v):
    cid = lax.axis_index("c")
    sid = lax.axis_index("s")
    wid = cid * NS + sid
    stripe = pl.ds(sid * RPS, RPS)

    pltpu.sync_copy(gidx.at[wid], gidx_v)
    pltpu.sync_copy(sidx.at[wid], sidx_v)
    pltpu.sync_copy(dis16.at[stripe], dis_own)
    pltpu.sync_copy(b1r, b1_v)
    _zero_acc(acc, sc_v, sid)

    @pl.loop(0, RPS // PB)
    def _(j):
        base = sid * RPS + j * PB
        _load4([(Dm.at[3, pl.ds(base, PB)], comb_v.at[0])], cs)

        @pl.loop(0, PB, unroll=4)
        def _(i):
            comb_v[1, i, :] = dis_own[j * PB + i, :] * comb_v[0, i, :]

        pltpu.sync_copy(comb_v.at[1], qscr.at[cid, pl.ds(base, PB)])

    plsc.subcore_barrier()

    _edge_loop(qscr.at[cid], gidx_v, sidx_v, w_s, acc, rows_v, sc_v,
               gs0, gs1, ss0, ss1, wid, w)
    plsc.subcore_barrier()
    pltpu.sync_copy(acc.at[stripe], pA.at[cid, stripe])
    _zero_acc(acc, sc_v, sid)
    pltpu.core_barrier(bar, core_axis_name="c")

    _build(pA, Dm.at[3], Dm.at[2], uA, qscr.at[cid], dis_own, comb_v, cs,
           cid, sid)
    plsc.subcore_barrier()
    _edge_loop(qscr.at[cid], gidx_v, sidx_v, w_s, acc, rows_v, sc_v,
               gs0, gs1, ss0, ss1, wid, w)
    plsc.subcore_barrier()
    pltpu.sync_copy(acc.at[stripe], pB.at[cid, stripe])
    _zero_acc(acc, sc_v, sid)
    pltpu.core_barrier(bar, core_axis_name="c")

    _build(pB, uA, Dm.at[1], uB, qscr.at[cid], dis_own, comb_v, cs,
           cid, sid)
    plsc.subcore_barrier()
    _edge_loop(qscr.at[cid], gidx_v, sidx_v, w_s, acc, rows_v, sc_v,
               gs0, gs1, ss0, ss1, wid, w)
    plsc.subcore_barrier()
    pltpu.sync_copy(acc.at[stripe], pC.at[cid, stripe])
    pltpu.core_barrier(bar, core_axis_name="c")

    @pl.when(cid == 0)
    def _():
        @pl.loop(0, RPS // PB)
        def _(j):
            base = sid * RPS + j * PB
            _load4([(pC.at[0, pl.ds(base, PB)], comb_v.at[0]),
                    (pC.at[1, pl.ds(base, PB)], comb_v.at[1]),
                    (uB.at[pl.ds(base, PB)], comb_v.at[2]),
                    (Dm.at[0, pl.ds(base, PB)], comb_v.at[3])], cs)
            bv = b1_v[0, :]

            @pl.loop(0, PB, unroll=4)
            def _(i):
                comb_v[0, i, :] = (
                    dis_own[j * PB + i, :]
                    * (comb_v[0, i, :] + comb_v[1, i, :])
                    + comb_v[2, i, :] + comb_v[3, i, :] + bv)

            pltpu.sync_copy(comb_v.at[0], out.at[pl.ds(base, PB)])


def _mm1_body(x_ref, w_ref, o_ref):
    o_ref[...] = jnp.dot(x_ref[...], w_ref[...],
                         preferred_element_type=jnp.float32)


def _mm1(xp, w):
    return pl.pallas_call(
        _mm1_body,
        grid=(NP_ // 1024,),
        in_specs=[pl.BlockSpec((1024, D), lambda i: (i, 0)),
                  pl.BlockSpec((D, 4 * U), lambda i: (0, 0))],
        out_specs=pl.BlockSpec((1024, 4 * U), lambda i: (i, 0)),
        out_shape=jax.ShapeDtypeStruct((NP_, 4 * U), jnp.float32),
    )(xp, w)


def _mm2_body(p0_ref, p1_ref, prev_ref, ck_ref, dis_ref, b_ref, w_ref, o_ref):
    z = (dis_ref[...] * (p0_ref[...] + p1_ref[...])
         + prev_ref[...] + ck_ref[...])
    h = jnp.maximum(z + b_ref[...], 0.0)
    o_ref[...] = jnp.dot(h, w_ref[...], preferred_element_type=jnp.float32)


def _mm2(p0, p1, prev, ck, dis16, b0, w):
    nspec = pl.BlockSpec((1024, U), lambda i: (i, 0))
    return pl.pallas_call(
        _mm2_body,
        grid=(NP_ // 1024,),
        in_specs=[nspec, nspec, nspec, nspec, nspec,
                  pl.BlockSpec((1, U), lambda i: (0, 0)),
                  pl.BlockSpec((U, 4 * U), lambda i: (0, 0))],
        out_specs=pl.BlockSpec((1024, 4 * U), lambda i: (i, 0)),
        out_shape=jax.ShapeDtypeStruct((NP_, 4 * U), jnp.float32),
    )(p0, p1, prev, ck, dis16, b0, w)


def kernel(x, edge_index, edge_weight, W0, b0, W1, b1):
    row = edge_index[0]
    col = edge_index[1]
    pad_e = EP - E
    rowp = jnp.concatenate(
        [row, jnp.zeros((pad_e,), jnp.int32)]).reshape(NW, NCH, CB)
    colp = jnp.concatenate(
        [col, jnp.zeros((pad_e,), jnp.int32)]).reshape(NW, NCH, CB)
    wp = jnp.concatenate(
        [edge_weight, jnp.zeros((pad_e,), jnp.float32)]).reshape(NW, NCH, CB)

    xp = jnp.pad(x, ((0, NP_ - N), (0, 0)))
    W0c = jnp.concatenate([W0[k * D:(k + 1) * D] for k in range(4)], axis=1)
    C = _mm1(xp, W0c)
    Csp = jnp.transpose(C.reshape(NP_, 4, U), (1, 0, 2))
    parts, t1v, dis16 = _layer1(Csp, rowp, colp, wp)[:3]

    W1c = jnp.concatenate([W1[k * U:(k + 1) * U] for k in range(4)], axis=1)
    Dm = _mm2(parts[0], parts[1], t1v, Csp[0], dis16,
              b0.reshape(1, U), W1c)
    Dsp = jnp.transpose(Dm.reshape(NP_, 4, U), (1, 0, 2))
    out = _layer2(Dsp, dis16, b1.reshape(1, U), rowp, colp, wp)[0]
    return out[:N]


# speedup vs baseline: 1.3032x; 1.1287x over previous
import functools

import jax
import jax.numpy as jnp
from jax import lax
from jax.experimental import pallas as pl
from jax.experimental.pallas import tpu as pltpu
from jax.experimental.pallas import tpu_sc as plsc

N = 10000
E = 320000
D = 128
U = 16

NC = 2
NS = 16
NW = NC * NS
CB = 128

NP_ = 10240
RPS = NP_ // NS
NCH = -(-E // (NW * CB))
EP = NW * NCH * CB
PB = 128

_vmesh = plsc.VectorSubcoreMesh(core_axis_name="c", subcore_axis_name="s")
_sc_params = pltpu.CompilerParams(use_tc_tiling_on_sc=False)

_NPU = jax.ShapeDtypeStruct((NP_, U), jnp.float32)
_2NPU = jax.ShapeDtypeStruct((NC, NP_, U), jnp.float32)


def _edge_loop(table, gidx_v, sidx_v, w_s, acc, rows_v, sc_v,
               gs, ss, wid, w):
    rb = tuple(rows_v.at[k] for k in range(4))
    sb = (sc_v.at[0], sc_v.at[1])

    def fire(g, b):
        if table is not None:
            pltpu.async_copy(table.at[gidx_v.at[g]], rb[b], gs[b])
        pltpu.async_copy(w.at[wid, g], w_s.at[b], gs[b])

    def scat_wait(g, b):
        pltpu.make_async_copy(sb[b], acc.at[sidx_v.at[g]], ss[b]).wait()

    def proc(g, gb, sc, drain):
        if table is not None:
            pltpu.make_async_copy(table.at[gidx_v.at[g]], rb[gb], gs[gb]).wait()
        pltpu.make_async_copy(w.at[wid, g], w_s.at[gb], gs[gb]).wait()
        if drain:
            scat_wait(g - 2, sc)

        @pl.loop(0, CB // U)
        def _(j):
            wv = w_s[gb, pl.ds(j * U, U)]
            for i in range(U):
                r = j * U + i
                if table is not None:
                    sb[sc][r, :] = rb[gb][r, :] * wv[i]
                else:
                    sb[sc][r, :] = lax.broadcast(wv[i], (U,))

        pltpu.async_copy(sb[sc], acc.at[sidx_v.at[g]], ss[sc], add=True)

    for k in range(4):
        fire(k, k)
    for k in range(4):
        proc(k, k, k % 2, k >= 2)
        fire(k + 4, k)

    NGRP = (NCH - 4) // 4
    @pl.loop(0, NGRP)
    def _(q):
        c0 = 4 * q + 4
        for k in range(4):
            c = c0 + k
            proc(c, k, k % 2, True)

            @pl.when(c + 4 < NCH)
            def _():
                fire(c + 4, k)

    for k in range(NCH - 4 * NGRP - 4):
        c = 4 * NGRP + 4 + k
        proc(c, k, c % 2, True)

    scat_wait(NCH - 2, (NCH - 2) % 2)
    scat_wait(NCH - 1, (NCH - 1) % 2)


def _zero_acc(acc, sc_v, sid):
    @pl.loop(0, PB)
    def _(i):
        sc_v[0, i, :] = jnp.zeros((U,), jnp.float32)

    @pl.loop(0, RPS // PB)
    def _(j):
        pltpu.sync_copy(sc_v.at[0, pl.ds(0, PB)],
                        acc.at[pl.ds(sid * RPS + j * PB, PB)])


def _load4(refs_tiles, cs):
    for src, dst in refs_tiles:
        pltpu.async_copy(src, dst, cs)
    for src, dst in refs_tiles:
        pltpu.make_async_copy(src, dst, cs).wait()


def _build(parts, prev, ck, tdst, qdst, dis_own, comb_v, cs, cid, sid):
    @pl.loop(0, RPS // PB)
    def _(j):
        base = sid * RPS + j * PB
        _load4([(parts.at[0, pl.ds(base, PB)], comb_v.at[0]),
                (parts.at[1, pl.ds(base, PB)], comb_v.at[1]),
                (prev.at[pl.ds(base, PB)], comb_v.at[2]),
                (ck.at[pl.ds(base, PB)], comb_v.at[3])], cs)

        @pl.loop(0, PB, unroll=4)
        def _(i):
            dv = dis_own[j * PB + i, :]
            t = (dv * (comb_v[0, i, :] + comb_v[1, i, :])
                 + comb_v[2, i, :] + comb_v[3, i, :])
            comb_v[0, i, :] = t
            comb_v[1, i, :] = dv * t

        pltpu.sync_copy(comb_v.at[1], qdst.at[pl.ds(base, PB)])

        @pl.when(cid == 0)
        def _():
            pltpu.sync_copy(comb_v.at[0], tdst.at[pl.ds(base, PB)])


_SC_SCRATCH = [
    pltpu.VMEM((NCH, CB), jnp.int32),
    pltpu.VMEM((NCH, CB), jnp.int32),
    pltpu.VMEM((4, CB, U), jnp.float32),
    pltpu.VMEM((2, CB, U), jnp.float32),
    pltpu.VMEM((4, CB), jnp.float32),
    pltpu.VMEM((4, PB, U), jnp.float32),
    pltpu.VMEM((RPS, U), jnp.float32),
    pltpu.VMEM_SHARED((NP_, U), jnp.float32),
    pltpu.SemaphoreType.DMA,
    pltpu.SemaphoreType.DMA,
    pltpu.SemaphoreType.DMA,
    pltpu.SemaphoreType.DMA,
    pltpu.SemaphoreType.DMA,
    pltpu.SemaphoreType.DMA,
    pltpu.SemaphoreType.DMA,
    pltpu.SemaphoreType.REGULAR,
]


def _rsqrt_newton(d):
    dm = jnp.maximum(d, 1e-12)
    bits = lax.bitcast_convert_type(dm, jnp.int32)
    y = lax.bitcast_convert_type(
        jnp.int32(0x5F3759DF) - lax.shift_right_logical(bits, 1),
        jnp.float32)
    hx = 0.5 * dm
    for _ in range(3):
        y = y * (1.5 - hx * y * y)
    return jnp.where(d > 0, y, 0.0)


@functools.partial(
    pl.kernel,
    out_type=(_2NPU, _NPU, _NPU, _2NPU, _2NPU, _NPU, _2NPU, _2NPU),
    mesh=_vmesh,
    compiler_params=_sc_params,
    scratch_types=_SC_SCRATCH,
)
def _layer1(C, gidx, sidx, w,
            parts, t_fin, dis16, qscr, degscr, tA, pA, pB,
            gidx_v, sidx_v, rows_v, sc_v, w_s, comb_v, dis_own, acc,
            gs0, gs1, gs2, gs3, ss0, ss1, cs, bar):
    cid = lax.axis_index("c")
    sid = lax.axis_index("s")
    wid = cid * NS + sid
    stripe = pl.ds(sid * RPS, RPS)

    pltpu.sync_copy(gidx.at[wid], gidx_v)
    pltpu.sync_copy(sidx.at[wid], sidx_v)
    _zero_acc(acc, sc_v, sid)
    plsc.subcore_barrier()

    _edge_loop(None, gidx_v, gidx_v, w_s, acc, rows_v, sc_v,
               (gs0, gs1, gs2, gs3), (ss0, ss1), wid, w)
    plsc.subcore_barrier()
    pltpu.sync_copy(acc.at[stripe], degscr.at[cid, stripe])
    _zero_acc(acc, sc_v, sid)
    pltpu.core_barrier(bar, core_axis_name="c")

    @pl.loop(0, RPS // PB)
    def _(j):
        base = sid * RPS + j * PB
        _load4([(degscr.at[0, pl.ds(base, PB)], comb_v.at[0]),
                (degscr.at[1, pl.ds(base, PB)], comb_v.at[1]),
                (C.at[3, pl.ds(base, PB)], comb_v.at[2])], cs)

        @pl.loop(0, PB, unroll=4)
        def _(i):
            y = _rsqrt_newton(comb_v[0, i, :] + comb_v[1, i, :])
            dis_own[j * PB + i, :] = y
            comb_v[3, i, :] = y * comb_v[2, i, :]

        pltpu.sync_copy(comb_v.at[3], qscr.at[cid, pl.ds(base, PB)])

        @pl.when(cid == 0)
        def _():
            pltpu.sync_copy(dis_own.at[pl.ds(j * PB, PB)],
                            dis16.at[pl.ds(base, PB)])

    plsc.subcore_barrier()

    _edge_loop(qscr.at[cid], gidx_v, sidx_v, w_s, acc, rows_v, sc_v,
               (gs0, gs1, gs2, gs3), (ss0, ss1), wid, w)
    plsc.subcore_barrier()
    pltpu.sync_copy(acc.at[stripe], pA.at[cid, stripe])
    _zero_acc(acc, sc_v, sid)
    pltpu.core_barrier(bar, core_axis_name="c")

    _build(pA, C.at[3], C.at[2], tA, qscr.at[cid], dis_own, comb_v, cs,
           cid, sid)
    plsc.subcore_barrier()
    _edge_loop(qscr.at[cid], gidx_v, sidx_v, w_s, acc, rows_v, sc_v,
               (gs0, gs1, gs2, gs3), (ss0, ss1), wid, w)
    plsc.subcore_barrier()
    pltpu.sync_copy(acc.at[stripe], pB.at[cid, stripe])
    _zero_acc(acc, sc_v, sid)
    pltpu.core_barrier(bar, core_axis_name="c")

    _build(pB, tA, C.at[1], t_fin, qscr.at[cid], dis_own, comb_v, cs,
           cid, sid)
    plsc.subcore_barrier()
    _edge_loop(qscr.at[cid], gidx_v, sidx_v, w_s, acc, rows_v, sc_v,
               (gs0, gs1, gs2, gs3), (ss0, ss1), wid, w)
    plsc.subcore_barrier()
    pltpu.sync_copy(acc.at[stripe], parts.at[cid, stripe])


@functools.partial(
    pl.kernel,
    out_type=(_NPU, _2NPU, _NPU, _NPU, _2NPU, _2NPU, _2NPU),
    mesh=_vmesh,
    compiler_params=_sc_params,
    scratch_types=_SC_SCRATCH + [pltpu.VMEM((1, U), jnp.float32)],
)
def _layer2(Dm, dis16, b1r, gidx, sidx, w,
            out, qscr, uA, uB, pA, pB, pC,
            gidx_v, sidx_v, rows_v, sc_v, w_s, comb_v, dis_own, acc,
            gs0, gs1, gs2, gs3, ss0, ss1, cs, bar, b1_v):
    cid = lax.axis_index("c")
    sid = lax.axis_index("s")
    wid = cid * NS + sid
    stripe = pl.ds(sid * RPS, RPS)

    pltpu.sync_copy(gidx.at[wid], gidx_v)
    pltpu.sync_copy(sidx.at[wid], sidx_v)
    pltpu.sync_copy(dis16.at[stripe], dis_own)
    pltpu.sync_copy(b1r, b1_v)
    _zero_acc(acc, sc_v, sid)

    @pl.loop(0, RPS // PB)
    def _(j):
        base = sid * RPS + j * PB
        _load4([(Dm.at[3, pl.ds(base, PB)], comb_v.at[0])], cs)

        @pl.loop(0, PB, unroll=4)
        def _(i):
            comb_v[1, i, :] = dis_own[j * PB + i, :] * comb_v[0, i, :]

        pltpu.sync_copy(comb_v.at[1], qscr.at[cid, pl.ds(base, PB)])

    plsc.subcore_barrier()

    _edge_loop(qscr.at[cid], gidx_v, sidx_v, w_s, acc, rows_v, sc_v,
               (gs0, gs1, gs2, gs3), (ss0, ss1), wid, w)
    plsc.subcore_barrier()
    pltpu.sync_copy(acc.at[stripe], pA.at[cid, stripe])
    _zero_acc(acc, sc_v, sid)
    pltpu.core_barrier(bar, core_axis_name="c")

    _build(pA, Dm.at[3], Dm.at[2], uA, qscr.at[cid], dis_own, comb_v, cs,
           cid, sid)
    plsc.subcore_barrier()
    _edge_loop(qscr.at[cid], gidx_v, sidx_v, w_s, acc, rows_v, sc_v,
               (gs0, gs1, gs2, gs3), (ss0, ss1), wid, w)
    plsc.subcore_barrier()
    pltpu.sync_copy(acc.at[stripe], pB.at[cid, stripe])
    _zero_acc(acc, sc_v, sid)
    pltpu.core_barrier(bar, core_axis_name="c")

    _build(pB, uA, Dm.at[1], uB, qscr.at[cid], dis_own, comb_v, cs,
           cid, sid)
    plsc.subcore_barrier()
    _edge_loop(qscr.at[cid], gidx_v, sidx_v, w_s, acc, rows_v, sc_v,
               (gs0, gs1, gs2, gs3), (ss0, ss1), wid, w)
    plsc.subcore_barrier()
    pltpu.sync_copy(acc.at[stripe], pC.at[cid, stripe])
    pltpu.core_barrier(bar, core_axis_name="c")

    @pl.when(cid == 0)
    def _():
        @pl.loop(0, RPS // PB)
        def _(j):
            base = sid * RPS + j * PB
            _load4([(pC.at[0, pl.ds(base, PB)], comb_v.at[0]),
                    (pC.at[1, pl.ds(base, PB)], comb_v.at[1]),
                    (uB.at[pl.ds(base, PB)], comb_v.at[2]),
                    (Dm.at[0, pl.ds(base, PB)], comb_v.at[3])], cs)
            bv = b1_v[0, :]

            @pl.loop(0, PB, unroll=4)
            def _(i):
                comb_v[0, i, :] = (
                    dis_own[j * PB + i, :]
                    * (comb_v[0, i, :] + comb_v[1, i, :])
                    + comb_v[2, i, :] + comb_v[3, i, :] + bv)

            pltpu.sync_copy(comb_v.at[0], out.at[pl.ds(base, PB)])


def _mm1_body(x_ref, w_ref, o_ref):
    o_ref[...] = jnp.dot(x_ref[...], w_ref[...],
                         preferred_element_type=jnp.float32)


def _mm1(xp, w):
    return pl.pallas_call(
        _mm1_body,
        grid=(NP_ // 1024,),
        in_specs=[pl.BlockSpec((1024, D), lambda i: (i, 0)),
                  pl.BlockSpec((D, 4 * U), lambda i: (0, 0))],
        out_specs=pl.BlockSpec((1024, 4 * U), lambda i: (i, 0)),
        out_shape=jax.ShapeDtypeStruct((NP_, 4 * U), jnp.float32),
    )(xp, w)


def _mm2_body(p0_ref, p1_ref, prev_ref, ck_ref, dis_ref, b_ref, w_ref, o_ref):
    z = (dis_ref[...] * (p0_ref[...] + p1_ref[...])
         + prev_ref[...] + ck_ref[...])
    h = jnp.maximum(z + b_ref[...], 0.0)
    o_ref[...] = jnp.dot(h, w_ref[...], preferred_element_type=jnp.float32)


def _mm2(p0, p1, prev, ck, dis16, b0, w):
    nspec = pl.BlockSpec((1024, U), lambda i: (i, 0))
    return pl.pallas_call(
        _mm2_body,
        grid=(NP_ // 1024,),
        in_specs=[nspec, nspec, nspec, nspec, nspec,
                  pl.BlockSpec((1, U), lambda i: (0, 0)),
                  pl.BlockSpec((U, 4 * U), lambda i: (0, 0))],
        out_specs=pl.BlockSpec((1024, 4 * U), lambda i: (i, 0)),
        out_shape=jax.ShapeDtypeStruct((NP_, 4 * U), jnp.float32),
    )(p0, p1, prev, ck, dis16, b0, w)


def kernel(x, edge_index, edge_weight, W0, b0, W1, b1):
    row = edge_index[0]
    col = edge_index[1]
    pad_e = EP - E
    rowp = jnp.concatenate(
        [row, jnp.zeros((pad_e,), jnp.int32)]).reshape(NW, NCH, CB)
    colp = jnp.concatenate(
        [col, jnp.zeros((pad_e,), jnp.int32)]).reshape(NW, NCH, CB)
    wp = jnp.concatenate(
        [edge_weight, jnp.zeros((pad_e,), jnp.float32)]).reshape(NW, NCH, CB)

    xp = jnp.pad(x, ((0, NP_ - N), (0, 0)))
    W0c = jnp.concatenate([W0[k * D:(k + 1) * D] for k in range(4)], axis=1)
    C = _mm1(xp, W0c)
    Csp = jnp.transpose(C.reshape(NP_, 4, U), (1, 0, 2))
    parts, t1v, dis16 = _layer1(Csp, rowp, colp, wp)[:3]

    W1c = jnp.concatenate([W1[k * U:(k + 1) * U] for k in range(4)], axis=1)
    Dm = _mm2(parts[0], parts[1], t1v, Csp[0], dis16,
              b0.reshape(1, U), W1c)
    Dsp = jnp.transpose(Dm.reshape(NP_, 4, U), (1, 0, 2))
    out = _layer2(Dsp, dis16, b1.reshape(1, U), rowp, colp, wp)[0]
    return out[:N]


# speedup vs baseline: 1.3446x; 1.0317x over previous
import functools

import jax
import jax.numpy as jnp
from jax import lax
from jax.experimental import pallas as pl
from jax.experimental.pallas import tpu as pltpu
from jax.experimental.pallas import tpu_sc as plsc

N = 10000
E = 320000
D = 128
U = 16

NC = 2
NS = 16
NW = NC * NS
CB = 128

NP_ = 10240
RPS = NP_ // NS
NCH = -(-E // (NW * CB))
EP = NW * NCH * CB
PB = 128
RING = 6

_vmesh = plsc.VectorSubcoreMesh(core_axis_name="c", subcore_axis_name="s")
_sc_params = pltpu.CompilerParams(use_tc_tiling_on_sc=False)

_NPU = jax.ShapeDtypeStruct((NP_, U), jnp.float32)
_2NPU = jax.ShapeDtypeStruct((NC, NP_, U), jnp.float32)


def _edge_loop(table, gidx_v, sidx_v, w_s, acc, rows_v, sc_v,
               gs, ss, wid, w):
    rb = tuple(rows_v.at[k] for k in range(RING))
    sb = (sc_v.at[0], sc_v.at[1])

    def fire(g, b):
        if table is not None:
            pltpu.async_copy(table.at[gidx_v.at[g]], rb[b], gs[b])
        pltpu.async_copy(w.at[wid, g], w_s.at[b], gs[b])

    def scat_wait(g, b):
        pltpu.make_async_copy(sb[b], acc.at[sidx_v.at[g]], ss[b]).wait()

    def proc(g, gb, sc, drain):
        if table is not None:
            pltpu.make_async_copy(table.at[gidx_v.at[g]], rb[gb], gs[gb]).wait()
        pltpu.make_async_copy(w.at[wid, g], w_s.at[gb], gs[gb]).wait()
        if drain:
            scat_wait(g - 2, sc)

        @pl.loop(0, CB // U)
        def _(j):
            wv = w_s[gb, pl.ds(j * U, U)]
            for i in range(U):
                r = j * U + i
                if table is not None:
                    sb[sc][r, :] = rb[gb][r, :] * wv[i]
                else:
                    sb[sc][r, :] = lax.broadcast(wv[i], (U,))

        pltpu.async_copy(sb[sc], acc.at[sidx_v.at[g]], ss[sc], add=True)

    for k in range(RING):
        fire(k, k)
    for k in range(RING):
        proc(k, k, k % 2, k >= 2)
        fire(k + RING, k)

    NGRP = (NCH - RING) // RING
    @pl.loop(0, NGRP)
    def _(q):
        c0 = RING * q + RING
        for k in range(RING):
            c = c0 + k
            proc(c, k, k % 2, True)

            @pl.when(c + RING < NCH)
            def _():
                fire(c + RING, k)

    for k in range(NCH - RING * NGRP - RING):
        c = RING * NGRP + RING + k
        proc(c, k, c % 2, True)

    scat_wait(NCH - 2, (NCH - 2) % 2)
    scat_wait(NCH - 1, (NCH - 1) % 2)


def _zero_acc(acc, sc_v, sid):
    @pl.loop(0, PB)
    def _(i):
        sc_v[0, i, :] = jnp.zeros((U,), jnp.float32)

    @pl.loop(0, RPS // PB)
    def _(j):
        pltpu.sync_copy(sc_v.at[0, pl.ds(0, PB)],
                        acc.at[pl.ds(sid * RPS + j * PB, PB)])


def _load4(refs_tiles, cs):
    for src, dst in refs_tiles:
        pltpu.async_copy(src, dst, cs)
    for src, dst in refs_tiles:
        pltpu.make_async_copy(src, dst, cs).wait()


def _build(parts, prev, ck, tdst, qdst, dis_own, comb_v, cs, cid, sid):
    @pl.loop(0, RPS // PB)
    def _(j):
        base = sid * RPS + j * PB
        _load4([(parts.at[0, pl.ds(base, PB)], comb_v.at[0]),
                (parts.at[1, pl.ds(base, PB)], comb_v.at[1]),
                (prev.at[pl.ds(base, PB)], comb_v.at[2]),
                (ck.at[pl.ds(base, PB)], comb_v.at[3])], cs)

        @pl.loop(0, PB, unroll=4)
        def _(i):
            dv = dis_own[j * PB + i, :]
            t = (dv * (comb_v[0, i, :] + comb_v[1, i, :])
                 + comb_v[2, i, :] + comb_v[3, i, :])
            comb_v[0, i, :] = t
            comb_v[1, i, :] = dv * t

        pltpu.sync_copy(comb_v.at[1], qdst.at[pl.ds(base, PB)])

        @pl.when(cid == 0)
        def _():
            pltpu.sync_copy(comb_v.at[0], tdst.at[pl.ds(base, PB)])


_SC_SCRATCH = [
    pltpu.VMEM((NCH, CB), jnp.int32),
    pltpu.VMEM((NCH, CB), jnp.int32),
    pltpu.VMEM((RING, CB, U), jnp.float32),
    pltpu.VMEM((2, CB, U), jnp.float32),
    pltpu.VMEM((RING, CB), jnp.float32),
    pltpu.VMEM((4, PB, U), jnp.float32),
    pltpu.VMEM((RPS, U), jnp.float32),
    pltpu.VMEM_SHARED((NP_, U), jnp.float32),
    pltpu.SemaphoreType.DMA,
    pltpu.SemaphoreType.DMA,
    pltpu.SemaphoreType.DMA,
    pltpu.SemaphoreType.DMA,
    pltpu.SemaphoreType.DMA,
    pltpu.SemaphoreType.DMA,
    pltpu.SemaphoreType.DMA,
    pltpu.SemaphoreType.DMA,
    pltpu.SemaphoreType.DMA,
    pltpu.SemaphoreType.REGULAR,
]


def _rsqrt_newton(d):
    dm = jnp.maximum(d, 1e-12)
    bits = lax.bitcast_convert_type(dm, jnp.int32)
    y = lax.bitcast_convert_type(
        jnp.int32(0x5F3759DF) - lax.shift_right_logical(bits, 1),
        jnp.float32)
    hx = 0.5 * dm
    for _ in range(3):
        y = y * (1.5 - hx * y * y)
    return jnp.where(d > 0, y, 0.0)


@functools.partial(
    pl.kernel,
    out_type=(_2NPU, _NPU, _NPU, _2NPU, _2NPU, _NPU, _2NPU, _2NPU),
    mesh=_vmesh,
    compiler_params=_sc_params,
    scratch_types=_SC_SCRATCH,
)
def _layer1(C, gidx, sidx, w,
            parts, t_fin, dis16, qscr, degscr, tA, pA, pB,
            gidx_v, sidx_v, rows_v, sc_v, w_s, comb_v, dis_own, acc,
            gs0, gs1, gs2, gs3, gs4, gs5, ss0, ss1, cs, bar):
    cid = lax.axis_index("c")
    sid = lax.axis_index("s")
    wid = cid * NS + sid
    stripe = pl.ds(sid * RPS, RPS)

    pltpu.sync_copy(gidx.at[wid], gidx_v)
    pltpu.sync_copy(sidx.at[wid], sidx_v)
    _zero_acc(acc, sc_v, sid)
    plsc.subcore_barrier()

    _edge_loop(None, gidx_v, gidx_v, w_s, acc, rows_v, sc_v,
               (gs0, gs1, gs2, gs3, gs4, gs5), (ss0, ss1), wid, w)
    plsc.subcore_barrier()
    pltpu.sync_copy(acc.at[stripe], degscr.at[cid, stripe])
    _zero_acc(acc, sc_v, sid)
    pltpu.core_barrier(bar, core_axis_name="c")

    @pl.loop(0, RPS // PB)
    def _(j):
        base = sid * RPS + j * PB
        _load4([(degscr.at[0, pl.ds(base, PB)], comb_v.at[0]),
                (degscr.at[1, pl.ds(base, PB)], comb_v.at[1]),
                (C.at[3, pl.ds(base, PB)], comb_v.at[2])], cs)

        @pl.loop(0, PB, unroll=4)
        def _(i):
            y = _rsqrt_newton(comb_v[0, i, :] + comb_v[1, i, :])
            dis_own[j * PB + i, :] = y
            comb_v[3, i, :] = y * comb_v[2, i, :]

        pltpu.sync_copy(comb_v.at[3], qscr.at[cid, pl.ds(base, PB)])

        @pl.when(cid == 0)
        def _():
            pltpu.sync_copy(dis_own.at[pl.ds(j * PB, PB)],
                            dis16.at[pl.ds(base, PB)])

    plsc.subcore_barrier()

    _edge_loop(qscr.at[cid], gidx_v, sidx_v, w_s, acc, rows_v, sc_v,
               (gs0, gs1, gs2, gs3, gs4, gs5), (ss0, ss1), wid, w)
    plsc.subcore_barrier()
    pltpu.sync_copy(acc.at[stripe], pA.at[cid, stripe])
    _zero_acc(acc, sc_v, sid)
    pltpu.core_barrier(bar, core_axis_name="c")

    _build(pA, C.at[3], C.at[2], tA, qscr.at[cid], dis_own, comb_v, cs,
           cid, sid)
    plsc.subcore_barrier()
    _edge_loop(qscr.at[cid], gidx_v, sidx_v, w_s, acc, rows_v, sc_v,
               (gs0, gs1, gs2, gs3, gs4, gs5), (ss0, ss1), wid, w)
    plsc.subcore_barrier()
    pltpu.sync_copy(acc.at[stripe], pB.at[cid, stripe])
    _zero_acc(acc, sc_v, sid)
    pltpu.core_barrier(bar, core_axis_name="c")

    _build(pB, tA, C.at[1], t_fin, qscr.at[cid], dis_own, comb_v, cs,
           cid, sid)
    plsc.subcore_barrier()
    _edge_loop(qscr.at[cid], gidx_v, sidx_v, w_s, acc, rows_v, sc_v,
               (gs0, gs1, gs2, gs3, gs4, gs5), (ss0, ss1), wid, w)
    plsc.subcore_barrier()
    pltpu.sync_copy(acc.at[stripe], parts.at[cid, stripe])


@functools.partial(
    pl.kernel,
    out_type=(_NPU, _2NPU, _NPU, _NPU, _2NPU, _2NPU, _2NPU),
    mesh=_vmesh,
    compiler_params=_sc_params,
    scratch_types=_SC_SCRATCH + [pltpu.VMEM((1, U), jnp.float32)],
)
def _layer2(Dm, dis16, b1r, gidx, sidx, w,
            out, qscr, uA, uB, pA, pB, pC,
            gidx_v, sidx_v, rows_v, sc_v, w_s, comb_v, dis_own, acc,
            gs0, gs1, gs2, gs3, gs4, gs5, ss0, ss1, cs, bar, b1_v):
    cid = lax.axis_index("c")
    sid = lax.axis_index("s")
    wid = cid * NS + sid
    stripe = pl.ds(sid * RPS, RPS)

    pltpu.sync_copy(gidx.at[wid], gidx_v)
    pltpu.sync_copy(sidx.at[wid], sidx_v)
    pltpu.sync_copy(dis16.at[stripe], dis_own)
    pltpu.sync_copy(b1r, b1_v)
    _zero_acc(acc, sc_v, sid)

    @pl.loop(0, RPS // PB)
    def _(j):
        base = sid * RPS + j * PB
        _load4([(Dm.at[3, pl.ds(base, PB)], comb_v.at[0])], cs)

        @pl.loop(0, PB, unroll=4)
        def _(i):
            comb_v[1, i, :] = dis_own[j * PB + i, :] * comb_v[0, i, :]

        pltpu.sync_copy(comb_v.at[1], qscr.at[cid, pl.ds(base, PB)])

    plsc.subcore_barrier()

    _edge_loop(qscr.at[cid], gidx_v, sidx_v, w_s, acc, rows_v, sc_v,
               (gs0, gs1, gs2, gs3, gs4, gs5), (ss0, ss1), wid, w)
    plsc.subcore_barrier()
    pltpu.sync_copy(acc.at[stripe], pA.at[cid, stripe])
    _zero_acc(acc, sc_v, sid)
    pltpu.core_barrier(bar, core_axis_name="c")

    _build(pA, Dm.at[3], Dm.at[2], uA, qscr.at[cid], dis_own, comb_v, cs,
           cid, sid)
    plsc.subcore_barrier()
    _edge_loop(qscr.at[cid], gidx_v, sidx_v, w_s, acc, rows_v, sc_v,
               (gs0, gs1, gs2, gs3, gs4, gs5), (ss0, ss1), wid, w)
    plsc.subcore_barrier()
    pltpu.sync_copy(acc.at[stripe], pB.at[cid, stripe])
    _zero_acc(acc, sc_v, sid)
    pltpu.core_barrier(bar, core_axis_name="c")

    _build(pB, uA, Dm.at[1], uB, qscr.at[cid], dis_own, comb_v, cs,
           cid, sid)
    plsc.subcore_barrier()
    _edge_loop(qscr.at[cid], gidx_v, sidx_v, w_s, acc, rows_v, sc_v,
               (gs0, gs1, gs2, gs3, gs4, gs5), (ss0, ss1), wid, w)
    plsc.subcore_barrier()
    pltpu.sync_copy(acc.at[stripe], pC.at[cid, stripe])
    pltpu.core_barrier(bar, core_axis_name="c")

    @pl.when(cid == 0)
    def _():
        @pl.loop(0, RPS // PB)
        def _(j):
            base = sid * RPS + j * PB
            _load4([(pC.at[0, pl.ds(base, PB)], comb_v.at[0]),
                    (pC.at[1, pl.ds(base, PB)], comb_v.at[1]),
                    (uB.at[pl.ds(base, PB)], comb_v.at[2]),
                    (Dm.at[0, pl.ds(base, PB)], comb_v.at[3])], cs)
            bv = b1_v[0, :]

            @pl.loop(0, PB, unroll=4)
            def _(i):
                comb_v[0, i, :] = (
                    dis_own[j * PB + i, :]
                    * (comb_v[0, i, :] + comb_v[1, i, :])
                    + comb_v[2, i, :] + comb_v[3, i, :] + bv)

            pltpu.sync_copy(comb_v.at[0], out.at[pl.ds(base, PB)])


def _mm1_body(x_ref, w_ref, o_ref):
    o_ref[...] = jnp.dot(x_ref[...], w_ref[...],
                         preferred_element_type=jnp.float32)


def _mm1(xp, w):
    return pl.pallas_call(
        _mm1_body,
        grid=(NP_ // 1024,),
        in_specs=[pl.BlockSpec((1024, D), lambda i: (i, 0)),
                  pl.BlockSpec((D, 4 * U), lambda i: (0, 0))],
        out_specs=pl.BlockSpec((1024, 4 * U), lambda i: (i, 0)),
        out_shape=jax.ShapeDtypeStruct((NP_, 4 * U), jnp.float32),
    )(xp, w)


def _mm2_body(p0_ref, p1_ref, prev_ref, ck_ref, dis_ref, b_ref, w_ref, o_ref):
    z = (dis_ref[...] * (p0_ref[...] + p1_ref[...])
         + prev_ref[...] + ck_ref[...])
    h = jnp.maximum(z + b_ref[...], 0.0)
    o_ref[...] = jnp.dot(h, w_ref[...], preferred_element_type=jnp.float32)


def _mm2(p0, p1, prev, ck, dis16, b0, w):
    nspec = pl.BlockSpec((1024, U), lambda i: (i, 0))
    return pl.pallas_call(
        _mm2_body,
        grid=(NP_ // 1024,),
        in_specs=[nspec, nspec, nspec, nspec, nspec,
                  pl.BlockSpec((1, U), lambda i: (0, 0)),
                  pl.BlockSpec((U, 4 * U), lambda i: (0, 0))],
        out_specs=pl.BlockSpec((1024, 4 * U), lambda i: (i, 0)),
        out_shape=jax.ShapeDtypeStruct((NP_, 4 * U), jnp.float32),
    )(p0, p1, prev, ck, dis16, b0, w)


def kernel(x, edge_index, edge_weight, W0, b0, W1, b1):
    row = edge_index[0]
    col = edge_index[1]
    pad_e = EP - E
    rowp = jnp.concatenate(
        [row, jnp.zeros((pad_e,), jnp.int32)]).reshape(NW, NCH, CB)
    colp = jnp.concatenate(
        [col, jnp.zeros((pad_e,), jnp.int32)]).reshape(NW, NCH, CB)
    wp = jnp.concatenate(
        [edge_weight, jnp.zeros((pad_e,), jnp.float32)]).reshape(NW, NCH, CB)

    xp = jnp.pad(x, ((0, NP_ - N), (0, 0)))
    W0c = jnp.concatenate([W0[k * D:(k + 1) * D] for k in range(4)], axis=1)
    C = _mm1(xp, W0c)
    Csp = jnp.transpose(C.reshape(NP_, 4, U), (1, 0, 2))
    parts, t1v, dis16 = _layer1(Csp, rowp, colp, wp)[:3]

    W1c = jnp.concatenate([W1[k * U:(k + 1) * U] for k in range(4)], axis=1)
    Dm = _mm2(parts[0], parts[1], t1v, Csp[0], dis16,
              b0.reshape(1, U), W1c)
    Dsp = jnp.transpose(Dm.reshape(NP_, 4, U), (1, 0, 2))
    out = _layer2(Dsp, dis16, b1.reshape(1, U), rowp, colp, wp)[0]
    return out[:N]
